# Initial kernel scaffold; baseline (speedup 1.0000x reference)
import jax
import jax.numpy as jnp
from jax.experimental import pallas as pl


def kernel(x, values, mix_w, mix_b):
    raise NotImplementedError("write your pallas kernel here")


# speedup vs baseline: 3589.2680x; 3589.2680x over previous
import functools

import numpy as np
import jax
import jax.numpy as jnp
from jax.experimental import pallas as pl
from jax.experimental.pallas import tpu as pltpu

IN_DIM = 256
OUT_DIM = 256
GRID_SIZE = 16

_KNOTS = np.linspace(-1.0, 1.0, GRID_SIZE).astype(np.float32)
_INV_H = (1.0 / (_KNOTS[1:] - _KNOTS[:-1] + 1e-8)).astype(np.float32)


def _kan_kernel(x_ref, vt_ref, wt_ref, b_ref, o_ref):
    vt = vt_ref[...]
    slopes = [
        (vt[s + 1:s + 2, :] - vt[s:s + 1, :]) * float(_INV_H[s])
        for s in range(GRID_SIZE - 1)
    ]
    db = [slopes[s] - slopes[s - 1] for s in range(1, GRID_SIZE - 1)]
    c0 = vt[0:1, :] - slopes[0] * float(_KNOTS[0])
    for s in range(1, GRID_SIZE - 1):
        c0 = c0 - db[s - 1] * float(_KNOTS[s])

    xb = jnp.clip(x_ref[...], -1.0, 1.0)
    acc = c0 + slopes[0] * xb
    for s in range(1, GRID_SIZE - 1):
        acc = acc + db[s - 1] * jnp.maximum(xb, float(_KNOTS[s]))

    o_ref[...] = (
        jnp.dot(acc, wt_ref[...], preferred_element_type=jnp.float32)
        + b_ref[...]
    )


@jax.jit
def kernel(x, values, mix_w, mix_b):
    B, D = x.shape
    TB = 1024
    grid = (B // TB,)
    vt = values.T
    wt = mix_w.T
    bias = mix_b.reshape(1, OUT_DIM)
    return pl.pallas_call(
        _kan_kernel,
        grid=grid,
        in_specs=[
            pl.BlockSpec((TB, D), lambda i: (i, 0)),
            pl.BlockSpec((GRID_SIZE, D), lambda i: (0, 0)),
            pl.BlockSpec((D, OUT_DIM), lambda i: (0, 0)),
            pl.BlockSpec((1, OUT_DIM), lambda i: (0, 0)),
        ],
        out_specs=pl.BlockSpec((TB, OUT_DIM), lambda i: (i, 0)),
        out_shape=jax.ShapeDtypeStruct((B, OUT_DIM), jnp.float32),
        compiler_params=pltpu.CompilerParams(
            dimension_semantics=("parallel",),
        ),
    )(x, vt, wt, bias)


# speedup vs baseline: 4880.0515x; 1.3596x over previous
import numpy as np
import jax
import jax.numpy as jnp
from jax.experimental import pallas as pl
from jax.experimental.pallas import tpu as pltpu

IN_DIM = 256
OUT_DIM = 256
GRID_SIZE = 16

_KNOTS = np.linspace(-1.0, 1.0, GRID_SIZE).astype(np.float32)
_INV_H = np.zeros((GRID_SIZE, 1), np.float32)
_INV_H[:-1, 0] = 1.0 / (_KNOTS[1:] - _KNOTS[:-1] + 1e-8)
_SCALE = np.float32((GRID_SIZE - 1) / 2.0)


def _kan_kernel(x_ref, vt_ref, invh_ref, knots_ref, wt_ref, b_ref, o_ref):
    vt = vt_ref[...]
    vt1 = pltpu.roll(vt, shift=GRID_SIZE - 1, axis=0)
    slope = (vt1 - vt) * invh_ref[...]
    icept = vt - knots_ref[...] * slope

    xb = jnp.clip(x_ref[...], -1.0, 1.0)
    u = jnp.clip((xb + 1.0) * _SCALE, 0.0, float(GRID_SIZE - 2))
    lo = u < 8.0
    s = u.astype(jnp.int32)
    s7 = jnp.bitwise_and(s, 7)
    a = jnp.where(
        lo,
        jnp.take_along_axis(icept[0:8, :], s7, axis=0, mode="promise_in_bounds"),
        jnp.take_along_axis(icept[8:16, :], s7, axis=0, mode="promise_in_bounds"),
    )
    b = jnp.where(
        lo,
        jnp.take_along_axis(slope[0:8, :], s7, axis=0, mode="promise_in_bounds"),
        jnp.take_along_axis(slope[8:16, :], s7, axis=0, mode="promise_in_bounds"),
    )
    acc = a + b * xb

    o_ref[...] = (
        jnp.dot(acc, wt_ref[...], preferred_element_type=jnp.float32)
        + b_ref[...]
    )


@jax.jit
def kernel(x, values, mix_w, mix_b):
    B, D = x.shape
    TB = 1024
    grid = (B // TB,)
    vt = values.T
    wt = mix_w.T
    bias = mix_b.reshape(1, OUT_DIM)
    invh = jnp.asarray(_INV_H)
    knots = jnp.asarray(_KNOTS[:, None])
    return pl.pallas_call(
        _kan_kernel,
        grid=grid,
        in_specs=[
            pl.BlockSpec((TB, D), lambda i: (i, 0)),
            pl.BlockSpec((GRID_SIZE, D), lambda i: (0, 0)),
            pl.BlockSpec((GRID_SIZE, 1), lambda i: (0, 0)),
            pl.BlockSpec((GRID_SIZE, 1), lambda i: (0, 0)),
            pl.BlockSpec((D, OUT_DIM), lambda i: (0, 0)),
            pl.BlockSpec((1, OUT_DIM), lambda i: (0, 0)),
        ],
        out_specs=pl.BlockSpec((TB, OUT_DIM), lambda i: (i, 0)),
        out_shape=jax.ShapeDtypeStruct((B, OUT_DIM), jnp.float32),
        compiler_params=pltpu.CompilerParams(
            dimension_semantics=("parallel",),
        ),
    )(x, vt, invh, knots, wt, bias)


# speedup vs baseline: 6238.2647x; 1.2783x over previous
import numpy as np
import jax
import jax.numpy as jnp
from jax.experimental import pallas as pl
from jax.experimental.pallas import tpu as pltpu

IN_DIM = 256
OUT_DIM = 256
GRID_SIZE = 16

_KNOTS = np.linspace(-1.0, 1.0, GRID_SIZE).astype(np.float32)
_INV_H = np.zeros((GRID_SIZE, 1), np.float32)
_INV_H[:-1, 0] = 1.0 / (_KNOTS[1:] - _KNOTS[:-1] + 1e-8)
_SCALE = np.float32((GRID_SIZE - 1) / 2.0)


def _kan_kernel(x_ref, vt_ref, invh_ref, knots_ref, wt_ref, b_ref, o_ref):
    vt = vt_ref[...]
    vt1 = pltpu.roll(vt, shift=GRID_SIZE - 1, axis=0)
    slope = (vt1 - vt) * invh_ref[...]
    icept = vt - knots_ref[...] * slope

    xb = jnp.clip(x_ref[...], -1.0, 1.0)
    u = jnp.clip((xb + 1.0) * _SCALE, 0.0, float(GRID_SIZE - 2))
    lo = u < 8.0
    s = u.astype(jnp.int32)
    s7 = jnp.bitwise_and(s, 7)
    a = jnp.where(
        lo,
        jnp.take_along_axis(icept[0:8, :], s7, axis=0, mode="promise_in_bounds"),
        jnp.take_along_axis(icept[8:16, :], s7, axis=0, mode="promise_in_bounds"),
    )
    b = jnp.where(
        lo,
        jnp.take_along_axis(slope[0:8, :], s7, axis=0, mode="promise_in_bounds"),
        jnp.take_along_axis(slope[8:16, :], s7, axis=0, mode="promise_in_bounds"),
    )
    acc = a + b * xb

    o_ref[...] = (
        jnp.dot(acc, wt_ref[...], preferred_element_type=jnp.float32)
        + b_ref[...]
    )


@jax.jit
def kernel(x, values, mix_w, mix_b):
    B, D = x.shape
    TB = 2048
    grid = (B // TB,)
    vt = values.T
    wt = mix_w.T
    bias = mix_b.reshape(1, OUT_DIM)
    invh = jnp.asarray(_INV_H)
    knots = jnp.asarray(_KNOTS[:, None])
    return pl.pallas_call(
        _kan_kernel,
        grid=grid,
        in_specs=[
            pl.BlockSpec((TB, D), lambda i: (i, 0)),
            pl.BlockSpec((GRID_SIZE, D), lambda i: (0, 0)),
            pl.BlockSpec((GRID_SIZE, 1), lambda i: (0, 0)),
            pl.BlockSpec((GRID_SIZE, 1), lambda i: (0, 0)),
            pl.BlockSpec((D, OUT_DIM), lambda i: (0, 0)),
            pl.BlockSpec((1, OUT_DIM), lambda i: (0, 0)),
        ],
        out_specs=pl.BlockSpec((TB, OUT_DIM), lambda i: (i, 0)),
        out_shape=jax.ShapeDtypeStruct((B, OUT_DIM), jnp.float32),
        compiler_params=pltpu.CompilerParams(
            dimension_semantics=("parallel",),
        ),
    )(x, vt, invh, knots, wt, bias)


# speedup vs baseline: 7180.5715x; 1.1511x over previous
import numpy as np
import jax
import jax.numpy as jnp
from jax.experimental import pallas as pl
from jax.experimental.pallas import tpu as pltpu

IN_DIM = 256
OUT_DIM = 256
GRID_SIZE = 16

_KNOTS = np.linspace(-1.0, 1.0, GRID_SIZE).astype(np.float32)
_INV_H = np.zeros((GRID_SIZE, 1), np.float32)
_INV_H[:-1, 0] = 1.0 / (_KNOTS[1:] - _KNOTS[:-1] + 1e-8)
_SCALE = np.float32((GRID_SIZE - 1) / 2.0)


def _kan_kernel(x_ref, vt_ref, invh_ref, knots_ref, wt_ref, b_ref, o_ref):
    vt = vt_ref[...]
    vt1 = pltpu.roll(vt, shift=GRID_SIZE - 1, axis=0)
    slope = (vt1 - vt) * invh_ref[...]
    icept = vt - knots_ref[...] * slope

    xb = jnp.clip(x_ref[...], -1.0, 1.0)
    u = jnp.clip((xb + 1.0) * _SCALE, 0.0, float(GRID_SIZE - 2))
    lo = u < 8.0
    s = u.astype(jnp.int32)
    s7 = jnp.bitwise_and(s, 7)
    a = jnp.where(
        lo,
        jnp.take_along_axis(icept[0:8, :], s7, axis=0, mode="promise_in_bounds"),
        jnp.take_along_axis(icept[8:16, :], s7, axis=0, mode="promise_in_bounds"),
    )
    b = jnp.where(
        lo,
        jnp.take_along_axis(slope[0:8, :], s7, axis=0, mode="promise_in_bounds"),
        jnp.take_along_axis(slope[8:16, :], s7, axis=0, mode="promise_in_bounds"),
    )
    acc = a + b * xb

    o_ref[...] = (
        jnp.dot(acc, wt_ref[...], preferred_element_type=jnp.float32)
        + b_ref[...]
    )


@jax.jit
def kernel(x, values, mix_w, mix_b):
    B, D = x.shape
    TB = 4096
    grid = (B // TB,)
    vt = values.T
    wt = mix_w.T
    bias = mix_b.reshape(1, OUT_DIM)
    invh = jnp.asarray(_INV_H)
    knots = jnp.asarray(_KNOTS[:, None])
    return pl.pallas_call(
        _kan_kernel,
        grid=grid,
        in_specs=[
            pl.BlockSpec((TB, D), lambda i: (i, 0)),
            pl.BlockSpec((GRID_SIZE, D), lambda i: (0, 0)),
            pl.BlockSpec((GRID_SIZE, 1), lambda i: (0, 0)),
            pl.BlockSpec((GRID_SIZE, 1), lambda i: (0, 0)),
            pl.BlockSpec((D, OUT_DIM), lambda i: (0, 0)),
            pl.BlockSpec((1, OUT_DIM), lambda i: (0, 0)),
        ],
        out_specs=pl.BlockSpec((TB, OUT_DIM), lambda i: (i, 0)),
        out_shape=jax.ShapeDtypeStruct((B, OUT_DIM), jnp.float32),
        compiler_params=pltpu.CompilerParams(
            dimension_semantics=("parallel",),
        ),
    )(x, vt, invh, knots, wt, bias)


# speedup vs baseline: 7565.5791x; 1.0536x over previous
import numpy as np
import jax
import jax.numpy as jnp
from jax.experimental import pallas as pl
from jax.experimental.pallas import tpu as pltpu

IN_DIM = 256
OUT_DIM = 256
GRID_SIZE = 16

_KNOTS = np.linspace(-1.0, 1.0, GRID_SIZE).astype(np.float32)
_INV_H = np.zeros((GRID_SIZE, 1), np.float32)
_INV_H[:-1, 0] = 1.0 / (_KNOTS[1:] - _KNOTS[:-1] + 1e-8)
_SCALE = np.float32((GRID_SIZE - 1) / 2.0)


def _kan_kernel(x_ref, vt_ref, invh_ref, knots_ref, wt_ref, b_ref, o_ref):
    vt = vt_ref[...]
    vt1 = pltpu.roll(vt, shift=GRID_SIZE - 1, axis=0)
    slope = (vt1 - vt) * invh_ref[...]
    icept = vt - knots_ref[...] * slope

    xb = jnp.clip(x_ref[...], -1.0, 1.0)
    u = jnp.clip((xb + 1.0) * _SCALE, 0.0, float(GRID_SIZE - 2))
    lo = u < 8.0
    s = u.astype(jnp.int32)
    s7 = jnp.bitwise_and(s, 7)
    a = jnp.where(
        lo,
        jnp.take_along_axis(icept[0:8, :], s7, axis=0, mode="promise_in_bounds"),
        jnp.take_along_axis(icept[8:16, :], s7, axis=0, mode="promise_in_bounds"),
    )
    b = jnp.where(
        lo,
        jnp.take_along_axis(slope[0:8, :], s7, axis=0, mode="promise_in_bounds"),
        jnp.take_along_axis(slope[8:16, :], s7, axis=0, mode="promise_in_bounds"),
    )
    acc = a + b * xb

    o_ref[...] = (
        jnp.dot(acc, wt_ref[...], preferred_element_type=jnp.float32)
        + b_ref[...]
    )


@jax.jit
def kernel(x, values, mix_w, mix_b):
    B, D = x.shape
    TB = 8192
    grid = (B // TB,)
    vt = values.T
    wt = mix_w.T
    bias = mix_b.reshape(1, OUT_DIM)
    invh = jnp.asarray(_INV_H)
    knots = jnp.asarray(_KNOTS[:, None])
    return pl.pallas_call(
        _kan_kernel,
        grid=grid,
        in_specs=[
            pl.BlockSpec((TB, D), lambda i: (i, 0)),
            pl.BlockSpec((GRID_SIZE, D), lambda i: (0, 0)),
            pl.BlockSpec((GRID_SIZE, 1), lambda i: (0, 0)),
            pl.BlockSpec((GRID_SIZE, 1), lambda i: (0, 0)),
            pl.BlockSpec((D, OUT_DIM), lambda i: (0, 0)),
            pl.BlockSpec((1, OUT_DIM), lambda i: (0, 0)),
        ],
        out_specs=pl.BlockSpec((TB, OUT_DIM), lambda i: (i, 0)),
        out_shape=jax.ShapeDtypeStruct((B, OUT_DIM), jnp.float32),
        compiler_params=pltpu.CompilerParams(
            dimension_semantics=("parallel",),
        ),
    )(x, vt, invh, knots, wt, bias)
